# baseline (speedup 1.0000x reference)
import functools

import jax
import jax.numpy as jnp
import numpy as np
from jax.experimental import pallas as pl
from jax.experimental.pallas import tpu as pltpu

_BB = 128
_GX = 16
_GP = 4


def _blkdiag_bf16(ablk, g, n):
    total = g * n
    acc = None
    for m in range(g):
        pw = ((m * n, total - (m + 1) * n), (m * n, total - (m + 1) * n))
        piece = jnp.pad(ablk[m], pw)
        acc = piece if acc is None else acc + piece
    return acc.astype(jnp.bfloat16)


def _gcn_kernel(x_ref, adj_ref, pose_ref, padj_ref,
                w1_ref, b1_ref, w3_ref, b3_ref,
                wp1_ref, bp1_ref, wp3_ref, bp3_ref,
                fcw_ref, fcb_ref, out_ref,
                w1s_ref, w3s_ref, wp1s_ref, wp3s_ref, msel_ref):
    f32 = jnp.float32
    bf16 = jnp.bfloat16

    @pl.when(pl.program_id(0) == 0)
    def _setup():
        w1s_ref[...] = w1_ref[...].astype(bf16)
        w3s_ref[...] = w3_ref[...].astype(bf16)
        wp1s_ref[0:90, :] = wp1_ref[...].astype(bf16)
        wp1s_ref[90:91, :] = bp1_ref[...].astype(bf16)
        wp3s_ref[...] = wp3_ref[...].astype(bf16)
        rows = jax.lax.broadcasted_iota(jnp.int32, (_BB, _BB * 34), 0)
        cols = jax.lax.broadcasted_iota(jnp.int32, (_BB, _BB * 34), 1)
        msel_ref[...] = jnp.where(cols // 34 == rows, 1.0 / 34.0, 0.0
                                  ).astype(bf16)

    ngx = _BB // _GX
    rgx = _GX * 8
    ngp = _BB // _GP
    rgp = _GP * 34

    def mix(amats, s, rg):
        sb = s.astype(bf16)
        parts = [jnp.dot(a, sb[k * rg:(k + 1) * rg],
                         preferred_element_type=f32)
                 for k, a in enumerate(amats)]
        return jnp.concatenate(parts, axis=0)

    adjb = jnp.transpose(adj_ref[...], (2, 0, 1))
    ax = [_blkdiag_bf16(adjb[k * _GX:(k + 1) * _GX], _GX, 8)
          for k in range(ngx)]
    padjb = jnp.transpose(padj_ref[...], (2, 0, 1))
    ap = [_blkdiag_bf16(padjb[k * _GP:(k + 1) * _GP], _GP, 34)
          for k in range(ngp)]

    xb = x_ref[...].astype(bf16)
    s1 = jnp.dot(xb, w1s_ref[...], preferred_element_type=f32)
    h1 = jnp.maximum(mix(ax, s1, rgx) + b1_ref[...], 0.0)
    s2 = jnp.dot(h1.astype(bf16), w3s_ref[...], preferred_element_type=f32)
    h2 = jnp.maximum(mix(ax, s2, rgx) + b3_ref[...], 0.0)
    hmean = jnp.mean(h2.reshape(_BB, 8, h2.shape[-1]), axis=1)

    pb = jnp.transpose(pose_ref[...], (2, 0, 1)).reshape(
        _BB * 34, pose_ref.shape[1])
    pm = mix(ap, pb, rgp)
    pm_aug = jnp.concatenate(
        [pm.astype(bf16),
         jnp.ones((pm.shape[0], 1), dtype=bf16)], axis=1)
    sp1 = jnp.dot(pm_aug, wp1s_ref[...], preferred_element_type=f32)
    p1 = jnp.maximum(sp1, 0.0)
    sp2 = jnp.dot(p1.astype(bf16), wp3s_ref[...], preferred_element_type=f32)
    p2 = jnp.maximum(mix(ap, sp2, rgp) + bp3_ref[...], 0.0)
    pmean = jnp.dot(msel_ref[...], p2.astype(bf16), preferred_element_type=f32)

    feat = jnp.concatenate([hmean, pmean], axis=1)
    fc_t = jax.lax.dot_general(fcw_ref[...].astype(bf16), feat.astype(bf16),
                               (((1,), (1,)), ((), ())),
                               preferred_element_type=f32)
    out_ref[...] = fc_t + fcb_ref[...]


@jax.jit
def kernel(x, adj, pose, pose_adj, W1, b1, W3, b3, Wp1, bp1, Wp3, bp3, fcW, fcb):
    B = x.shape[0]
    bb = _BB
    nblk = B // bb

    xf = x.reshape(B * 8, x.shape[-1])
    adj_t = jnp.transpose(adj, (1, 2, 0))
    pose_t = jnp.transpose(pose, (1, 2, 0))
    padj_t = jnp.transpose(pose_adj, (1, 2, 0))
    fcw_t = fcW.T

    bf16 = jnp.bfloat16
    b1r, b3r = b1.reshape(1, -1), b3.reshape(1, -1)
    bp1r, bp3r = bp1.reshape(1, -1), bp3.reshape(1, -1)
    fcbr = fcb.reshape(-1, 1)

    const2 = lambda i: (0, 0)

    out = pl.pallas_call(
        _gcn_kernel,
        grid=(nblk,),
        in_specs=[
            pl.BlockSpec((bb * 8, x.shape[-1]), lambda i: (i, 0)),
            pl.BlockSpec((8, 8, bb), lambda i: (0, 0, i)),
            pl.BlockSpec((34, pose.shape[-1], bb), lambda i: (0, 0, i)),
            pl.BlockSpec((34, 34, bb), lambda i: (0, 0, i)),
            pl.BlockSpec(W1.shape, const2),
            pl.BlockSpec(b1r.shape, const2),
            pl.BlockSpec(W3.shape, const2),
            pl.BlockSpec(b3r.shape, const2),
            pl.BlockSpec(Wp1.shape, const2),
            pl.BlockSpec(bp1r.shape, const2),
            pl.BlockSpec(Wp3.shape, const2),
            pl.BlockSpec(bp3r.shape, const2),
            pl.BlockSpec(fcw_t.shape, const2),
            pl.BlockSpec(fcbr.shape, const2),
        ],
        out_specs=pl.BlockSpec((fcW.shape[-1], bb), lambda i: (0, i)),
        out_shape=jax.ShapeDtypeStruct((fcW.shape[-1], B), jnp.float32),
        scratch_shapes=[
            pltpu.VMEM(W1.shape, bf16),
            pltpu.VMEM(W3.shape, bf16),
            pltpu.VMEM((Wp1.shape[0] + 1, Wp1.shape[1]), bf16),
            pltpu.VMEM(Wp3.shape, bf16),
            pltpu.VMEM((bb, bb * 34), bf16),
        ],
    )(xf, adj_t, pose_t, padj_t, W1, b1r, W3, b3r, Wp1, bp1r, Wp3, bp3r,
      fcw_t, fcbr)
    return out.T
